# speedup vs baseline: 3.0980x; 3.0980x over previous
import jax
import jax.numpy as jnp
from jax.experimental import pallas as pl
from jax.experimental.pallas import tpu as pltpu
from jax.experimental.pallas import tpu_sc as plsc

_WINDOW = 128


def _make_gather(num_indices: int, dim: int, dtype):
    mesh = plsc.VectorSubcoreMesh(
        core_axis_name="core", subcore_axis_name="subcore"
    )

    @pl.kernel(
        out_type=jax.ShapeDtypeStruct((num_indices, dim), dtype),
        mesh=mesh,
    )
    def gather_kernel(w_hbm, i_hbm, o_hbm):
        def body(i_vmem, o_vmem):
            pltpu.sync_copy(w_hbm.at[i_vmem.at[0]], o_vmem)

        pltpu.emit_pipeline(
            body,
            grid=(num_indices // _WINDOW,),
            in_specs=[
                pl.BlockSpec((1, _WINDOW), index_map=lambda i: (0, i))
            ],
            out_specs=[
                pl.BlockSpec((_WINDOW, dim), index_map=lambda i: (i, 0))
            ],
            core_axis_name=("core", "subcore"),
            dimension_semantics=(pltpu.PARALLEL,),
        )(i_hbm, o_hbm)

    return gather_kernel


def kernel(x, weight):
    batch, hist = x.shape
    num_indices = batch * hist
    dim = weight.shape[1]
    idx = x.reshape(1, num_indices).astype(jnp.int32)
    out = _make_gather(num_indices, dim, weight.dtype)(weight, idx)
    return out.reshape(batch, hist, dim)


# speedup vs baseline: 3.1321x; 1.0110x over previous
import functools

import jax
import jax.numpy as jnp
from jax import lax
from jax.experimental import pallas as pl
from jax.experimental.pallas import tpu as pltpu
from jax.experimental.pallas import tpu_sc as plsc

_CH = 128
_NCH = 50
_NW = 32
_D = 128


def _make_gather(dtype):
    mesh = plsc.VectorSubcoreMesh(
        core_axis_name="core", subcore_axis_name="subcore"
    )
    num_rows = _NW * _NCH * _CH

    @functools.partial(
        pl.kernel,
        mesh=mesh,
        out_type=jax.ShapeDtypeStruct((num_rows, _D), dtype),
        scratch_types=[
            pltpu.VMEM((_NCH, _CH), jnp.int32),
            pltpu.VMEM((_CH, _D), dtype),
            pltpu.VMEM((_CH, _D), dtype),
            pltpu.SemaphoreType.DMA,
            pltpu.SemaphoreType.DMA,
            pltpu.SemaphoreType.DMA,
            pltpu.SemaphoreType.DMA,
        ],
    )
    def gather_kernel(w_hbm, i_hbm, o_hbm, idx_v, buf0, buf1, gs0, gs1,
                      os0, os1):
        wid = lax.axis_index("subcore") * 2 + lax.axis_index("core")
        base = wid * (_NCH * _CH)
        pltpu.sync_copy(i_hbm.at[wid], idx_v)

        bufs = (buf0, buf1)
        gsems = (gs0, gs1)
        osems = (os0, os1)

        def start_gather(j, b):
            pltpu.async_copy(w_hbm.at[idx_v.at[j]], bufs[b], gsems[b])

        def wait_gather(b):
            pltpu.make_async_copy(
                w_hbm.at[idx_v.at[0]], bufs[b], gsems[b]
            ).wait()

        def start_out(j, b):
            pltpu.async_copy(
                bufs[b], o_hbm.at[pl.ds(base + j * _CH, _CH)], osems[b]
            )

        def wait_out(b):
            pltpu.make_async_copy(
                bufs[b], o_hbm.at[pl.ds(base, _CH)], osems[b]
            ).wait()

        start_gather(0, 0)

        @pl.loop(0, _NCH // 2)
        def _(p):
            j0 = p * 2

            wait_gather(0)

            @pl.when(p > 0)
            def _():
                wait_out(1)

            start_gather(j0 + 1, 1)
            start_out(j0, 0)

            wait_gather(1)

            @pl.when(p < _NCH // 2 - 1)
            def _():
                wait_out(0)
                start_gather(j0 + 2, 0)

            start_out(j0 + 1, 1)

        wait_out(0)
        wait_out(1)

    return gather_kernel


def kernel(x, weight):
    batch, hist = x.shape
    dim = weight.shape[1]
    idx = x.reshape(_NW, _NCH, _CH).astype(jnp.int32)
    out = _make_gather(weight.dtype)(weight, idx)
    return out.reshape(batch, hist, dim)


# speedup vs baseline: 5.9562x; 1.9017x over previous
import functools

import jax
import jax.numpy as jnp
from jax import lax
from jax.experimental import pallas as pl
from jax.experimental.pallas import tpu as pltpu
from jax.experimental.pallas import tpu_sc as plsc

_NW = 32
_BW = 2
_NBUF = 4


def _make_gather(dtype, batch, hist, dim):
    mesh = plsc.VectorSubcoreMesh(
        core_axis_name="core", subcore_axis_name="subcore"
    )
    bpw = batch // _NW
    nwin = bpw // _BW
    rows = _BW * hist

    @functools.partial(
        pl.kernel,
        mesh=mesh,
        out_type=jax.ShapeDtypeStruct((batch, hist, dim), dtype),
        scratch_types=[
            pltpu.VMEM((nwin, rows), jnp.int32),
        ]
        + [pltpu.VMEM((rows, dim), dtype) for _ in range(_NBUF)]
        + [pltpu.SemaphoreType.DMA for _ in range(2 * _NBUF)],
    )
    def gather_kernel(w_hbm, x_hbm, o_hbm, idx_v, *rest):
        bufs = rest[:_NBUF]
        gsems = rest[_NBUF:2 * _NBUF]
        osems = rest[2 * _NBUF:]

        wid = lax.axis_index("subcore") * 2 + lax.axis_index("core")
        base = wid * bpw
        pltpu.sync_copy(x_hbm.at[wid], idx_v)

        def start_gather(j, b):
            pltpu.async_copy(w_hbm.at[idx_v.at[j]], bufs[b], gsems[b])

        def wait_gather(b):
            pltpu.make_async_copy(
                w_hbm.at[idx_v.at[0]], bufs[b], gsems[b]
            ).wait()

        def start_out(j, b):
            b0 = base + j * _BW
            for k in range(_BW):
                pltpu.async_copy(
                    bufs[b].at[pl.ds(k * hist, hist)],
                    o_hbm.at[b0 + k],
                    osems[b],
                )

        def wait_out(b):
            for _ in range(_BW):
                pltpu.make_async_copy(
                    bufs[b].at[pl.ds(0, hist)], o_hbm.at[0], osems[b]
                ).wait()

        for b in range(_NBUF - 1):
            start_gather(b, b)

        @pl.loop(0, nwin // _NBUF)
        def _(p):
            for b in range(_NBUF):
                j = p * _NBUF + b
                wait_gather(b)
                start_out(j, b)
                gb = (b + _NBUF - 1) % _NBUF
                g = j + _NBUF - 1
                if b == 0:
                    @pl.when(p > 0)
                    def _():
                        wait_out(gb)
                    start_gather(g, gb)
                else:
                    wait_out(gb)

                    @pl.when(g < nwin)
                    def _():
                        start_gather(g, gb)

        wait_out((nwin - 1) % _NBUF)

    return gather_kernel


def kernel(x, weight):
    batch, hist = x.shape
    dim = weight.shape[1]
    bpw = batch // _NW
    idx = x.astype(jnp.int32).reshape(_NW, bpw // _BW, _BW * hist)
    return _make_gather(weight.dtype, batch, hist, dim)(weight, idx)
